# Initial kernel scaffold; baseline (speedup 1.0000x reference)
import jax
import jax.numpy as jnp
from jax.experimental import pallas as pl


def kernel(x, edge_index, Wl1, bl1, Wr1, bnw1, bnb1, Wl2, bl2, Wr2, bnw2, bnb2, Wl3, bl3, Wr3, bnw3, bnb3, Wc1, bc1, Wc2, bc2):
    raise NotImplementedError("write your pallas kernel here")


# speedup vs baseline: 3.1491x; 3.1491x over previous
import functools

import jax
import jax.numpy as jnp
from jax import lax
from jax.experimental import pallas as pl
from jax.experimental.pallas import tpu as pltpu
from jax.experimental.pallas import tpu_sc as plsc

N = 10000
E = 320000
D = 128
H = 128
EPS = 1e-5

NC = 2
NS = 16
NW = NC * NS
CHUNK = 128
MAXC = 80
NCHUNK = NW * MAXC
EPAD = NCHUNK * CHUNK - E
NP = 10240
ROWS_PER_TILE = NP // NS
ZROWS = 64


def _sc_mesh():
  return plsc.VectorSubcoreMesh(core_axis_name="c", subcore_axis_name="s")


def _make_sc_agg():
  scratch = [
      pltpu.VMEM((MAXC, CHUNK), jnp.int32),
      pltpu.VMEM((MAXC, CHUNK), jnp.int32),
      pltpu.VMEM((CHUNK, D), jnp.float32),
      pltpu.VMEM((ZROWS, D), jnp.float32),
      pltpu.VMEM_SHARED((NP, D), jnp.float32),
  ]

  def body(h_hbm, e_hbm, z_hbm, agg_hbm, srcv, dstv, rows, zv, acc):
    cid = lax.axis_index("c")
    sid = lax.axis_index("s")
    wid = cid * NS + sid
    row0 = sid * ROWS_PER_TILE

    pltpu.sync_copy(z_hbm, zv)

    @pl.loop(0, ROWS_PER_TILE // ZROWS)
    def _(j):
      pltpu.sync_copy(zv, acc.at[pl.ds(row0 + j * ZROWS, ZROWS)])

    ch0 = wid * MAXC
    pltpu.sync_copy(e_hbm.at[0, pl.ds(ch0, MAXC)], srcv)
    pltpu.sync_copy(e_hbm.at[1, pl.ds(ch0, MAXC)], dstv)

    plsc.subcore_barrier()

    @pl.loop(0, MAXC)
    def _(c):
      pltpu.sync_copy(h_hbm.at[srcv.at[c]], rows)
      pltpu.sync_copy(rows, acc.at[dstv.at[c]], add=True)

    plsc.subcore_barrier()

    pltpu.sync_copy(acc.at[pl.ds(row0, ROWS_PER_TILE)],
                    agg_hbm.at[cid, pl.ds(row0, ROWS_PER_TILE)])

  return pl.kernel(body, mesh=_sc_mesh(),
                   out_type=jax.ShapeDtypeStruct((NC, NP, D), jnp.float32),
                   scratch_types=scratch)


def _make_sc_counts():
  scratch = [
      pltpu.VMEM((MAXC, CHUNK), jnp.int32),
      pltpu.VMEM((CHUNK, D), jnp.float32),
      pltpu.VMEM((ZROWS, D), jnp.float32),
      pltpu.VMEM_SHARED((NP, D), jnp.float32),
  ]

  def body(e_hbm, zc_hbm, ones_hbm, cnt_hbm, dstv, onesv, zcv, cacc):
    cid = lax.axis_index("c")
    sid = lax.axis_index("s")
    wid = cid * NS + sid
    row0 = sid * ROWS_PER_TILE

    pltpu.sync_copy(zc_hbm, zcv)
    pltpu.sync_copy(ones_hbm, onesv)

    @pl.loop(0, ROWS_PER_TILE // ZROWS)
    def _(j):
      pltpu.sync_copy(zcv, cacc.at[pl.ds(row0 + j * ZROWS, ZROWS)])

    ch0 = wid * MAXC
    pltpu.sync_copy(e_hbm.at[1, pl.ds(ch0, MAXC)], dstv)

    plsc.subcore_barrier()

    @pl.loop(0, MAXC)
    def _(c):
      pltpu.sync_copy(onesv, cacc.at[dstv.at[c]], add=True)

    plsc.subcore_barrier()

    pltpu.sync_copy(cacc.at[pl.ds(row0, ROWS_PER_TILE)],
                    cnt_hbm.at[cid, pl.ds(row0, ROWS_PER_TILE)])

  return pl.kernel(body, mesh=_sc_mesh(),
                   out_type=jax.ShapeDtypeStruct((NC, NP, D), jnp.float32),
                   scratch_types=scratch)


def _sc_agg(h, e3, zrow):
  return _make_sc_agg()(h, e3, zrow)


def _sc_counts(e3, zcnt, ones):
  return _make_sc_counts()(e3, zcnt, ones)


RB = 1000
GRID = N // RB


def _linear_body(agg_ref, cnt_ref, h_ref, wl_ref, bl_ref, wr_ref,
                 y_ref, st_ref, acc_ref):
  i = pl.program_id(0)
  agg = agg_ref[0] + agg_ref[1]
  cnt = cnt_ref[0, :, :1] + cnt_ref[1, :, :1]
  mean = agg / jnp.maximum(cnt, 1.0)
  y = (jnp.dot(mean, wl_ref[...], preferred_element_type=jnp.float32)
       + jnp.dot(h_ref[...], wr_ref[...], preferred_element_type=jnp.float32)
       + bl_ref[...])
  y_ref[...] = y

  @pl.when(i == 0)
  def _():
    acc_ref[...] = jnp.zeros_like(acc_ref)

  acc_ref[0:1, :] += jnp.sum(y, axis=0, keepdims=True)
  acc_ref[1:2, :] += jnp.sum(y * y, axis=0, keepdims=True)
  st_ref[...] = acc_ref[...]


def _linear(aggp, cntp, h, Wl, bl, Wr):
  return pl.pallas_call(
      _linear_body,
      grid=(GRID,),
      in_specs=[
          pl.BlockSpec((NC, RB, D), lambda i: (0, i, 0)),
          pl.BlockSpec((NC, RB, D), lambda i: (0, i, 0)),
          pl.BlockSpec((RB, D), lambda i: (i, 0)),
          pl.BlockSpec((D, H), lambda i: (0, 0)),
          pl.BlockSpec((1, H), lambda i: (0, 0)),
          pl.BlockSpec((D, H), lambda i: (0, 0)),
      ],
      out_specs=[
          pl.BlockSpec((RB, H), lambda i: (i, 0)),
          pl.BlockSpec((8, 128), lambda i: (0, 0)),
      ],
      out_shape=[
          jax.ShapeDtypeStruct((N, H), jnp.float32),
          jax.ShapeDtypeStruct((8, 128), jnp.float32),
      ],
      scratch_shapes=[pltpu.VMEM((8, 128), jnp.float32)],
  )(aggp, cntp, h, Wl, bl, Wr)


def _norm_body(y_ref, st_ref, w_ref, b_ref, h_ref):
  mu = st_ref[0:1, :] * (1.0 / N)
  var = st_ref[1:2, :] * (1.0 / N) - mu * mu
  inv = lax.rsqrt(var + EPS)
  h = (y_ref[...] - mu) * (inv * w_ref[...]) + b_ref[...]
  h_ref[...] = jnp.maximum(h, 0.0)


def _norm_relu(y, st, w, b):
  return pl.pallas_call(
      _norm_body,
      grid=(GRID,),
      in_specs=[
          pl.BlockSpec((RB, H), lambda i: (i, 0)),
          pl.BlockSpec((8, 128), lambda i: (0, 0)),
          pl.BlockSpec((1, H), lambda i: (0, 0)),
          pl.BlockSpec((1, H), lambda i: (0, 0)),
      ],
      out_specs=pl.BlockSpec((RB, H), lambda i: (i, 0)),
      out_shape=jax.ShapeDtypeStruct((N, H), jnp.float32),
  )(y, st, w, b)


def _head_body(y_ref, st_ref, w_ref, b_ref, wc1_ref, bc1_ref, wc2_ref,
               bc2_ref, o_ref):
  mu = st_ref[0:1, :] * (1.0 / N)
  var = st_ref[1:2, :] * (1.0 / N) - mu * mu
  inv = lax.rsqrt(var + EPS)
  h = (y_ref[...] - mu) * (inv * w_ref[...]) + b_ref[...]
  t = jnp.dot(h, wc1_ref[...], preferred_element_type=jnp.float32)
  t = jnp.maximum(t + bc1_ref[...], 0.0)
  o = jnp.sum(t * wc2_ref[...], axis=1, keepdims=True) + bc2_ref[...]
  o_ref[...] = o


def _head(y, st, w, b, Wc1, bc1, wc2r, bc2r):
  return pl.pallas_call(
      _head_body,
      grid=(GRID,),
      in_specs=[
          pl.BlockSpec((RB, H), lambda i: (i, 0)),
          pl.BlockSpec((8, 128), lambda i: (0, 0)),
          pl.BlockSpec((1, H), lambda i: (0, 0)),
          pl.BlockSpec((1, H), lambda i: (0, 0)),
          pl.BlockSpec((H, H // 2), lambda i: (0, 0)),
          pl.BlockSpec((1, H // 2), lambda i: (0, 0)),
          pl.BlockSpec((1, H // 2), lambda i: (0, 0)),
          pl.BlockSpec((1, 1), lambda i: (0, 0)),
      ],
      out_specs=pl.BlockSpec((RB, 1), lambda i: (i, 0)),
      out_shape=jax.ShapeDtypeStruct((N, 1), jnp.float32),
  )(y, st, w, b, Wc1, bc1, wc2r, bc2r)


def kernel(x, edge_index, Wl1, bl1, Wr1, bnw1, bnb1, Wl2, bl2, Wr2, bnw2,
           bnb2, Wl3, bl3, Wr3, bnw3, bnb3, Wc1, bc1, Wc2, bc2):
  pad_src = jnp.zeros((EPAD,), jnp.int32)
  pad_dst = N + (jnp.arange(EPAD, dtype=jnp.int32) % (NP - N))
  e3 = jnp.concatenate(
      [edge_index, jnp.stack([pad_src, pad_dst])], axis=1
  ).reshape(2, NCHUNK, CHUNK)
  zrow = jnp.zeros((ZROWS, D), jnp.float32)
  ones = jnp.ones((CHUNK, D), jnp.float32)

  cntp = _sc_counts(e3, zrow, ones)
  agg1 = _sc_agg(x, e3, zrow)
  y1, st1 = _linear(agg1, cntp, x, Wl1, bl1.reshape(1, H), Wr1)
  h1 = _norm_relu(y1, st1, bnw1.reshape(1, H), bnb1.reshape(1, H))

  agg2 = _sc_agg(h1, e3, zrow)
  y2, st2 = _linear(agg2, cntp, h1, Wl2, bl2.reshape(1, H), Wr2)
  h2 = _norm_relu(y2, st2, bnw2.reshape(1, H), bnb2.reshape(1, H))

  agg3 = _sc_agg(h2, e3, zrow)
  y3, st3 = _linear(agg3, cntp, h2, Wl3, bl3.reshape(1, H), Wr3)
  out = _head(y3, st3, bnw3.reshape(1, H), bnb3.reshape(1, H),
              Wc1, bc1.reshape(1, H // 2), Wc2.reshape(1, H // 2),
              bc2.reshape(1, 1))
  return out[:, 0]


# speedup vs baseline: 3.4838x; 1.1063x over previous
import functools

import jax
import jax.numpy as jnp
from jax import lax
from jax.experimental import pallas as pl
from jax.experimental.pallas import tpu as pltpu
from jax.experimental.pallas import tpu_sc as plsc

N = 10000
E = 320000
D = 128
H = 128
EPS = 1e-5

NC = 2
NS = 16
NW = NC * NS
CHUNK = 128
MAXC = 80
NCHUNK = NW * MAXC
EPAD = NCHUNK * CHUNK - E
NP = 10240
ROWS_PER_TILE = NP // NS
ZROWS = 16
IDXH = MAXC // 2


def _sc_mesh():
  return plsc.VectorSubcoreMesh(core_axis_name="c", subcore_axis_name="s")


def _make_sc_agg():
  scratch = [
      pltpu.VMEM((IDXH, CHUNK), jnp.int32),
      pltpu.VMEM((IDXH, CHUNK), jnp.int32),
      pltpu.VMEM((CHUNK, D), jnp.float32),
      pltpu.VMEM((CHUNK, D), jnp.float32),
      pltpu.VMEM((ZROWS, D), jnp.float32),
      pltpu.SemaphoreType.DMA,
      pltpu.SemaphoreType.DMA,
      pltpu.VMEM_SHARED((NP, D), jnp.float32),
  ]

  def body(h_hbm, e_hbm, z_hbm, agg_hbm, srcv, dstv, rows0, rows1, zv,
           sg0, sg1, acc):
    cid = lax.axis_index("c")
    sid = lax.axis_index("s")
    wid = cid * NS + sid
    row0 = sid * ROWS_PER_TILE

    pltpu.sync_copy(z_hbm, zv)

    @pl.loop(0, ROWS_PER_TILE // ZROWS)
    def _(j):
      pltpu.sync_copy(zv, acc.at[pl.ds(row0 + j * ZROWS, ZROWS)])

    plsc.subcore_barrier()

    bufs = ((rows0, sg0), (rows1, sg1))

    for half in range(2):
      ch0 = wid * MAXC + half * IDXH
      pltpu.sync_copy(e_hbm.at[0, pl.ds(ch0, IDXH)], srcv)
      pltpu.sync_copy(e_hbm.at[1, pl.ds(ch0, IDXH)], dstv)

      for b, (buf, sem) in enumerate(bufs):
        pltpu.async_copy(h_hbm.at[srcv.at[b]], buf, sem)

      @pl.loop(0, IDXH // 2)
      def _(g):
        for b, (buf, sem) in enumerate(bufs):
          c = 2 * g + b
          pltpu.make_async_copy(h_hbm.at[pl.ds(0, CHUNK)], buf, sem).wait()
          pltpu.sync_copy(buf, acc.at[dstv.at[c]], add=True)

          @pl.when(c + 2 < IDXH)
          def _():
            pltpu.async_copy(h_hbm.at[srcv.at[c + 2]], buf, sem)

    plsc.subcore_barrier()

    pltpu.sync_copy(acc.at[pl.ds(row0, ROWS_PER_TILE)],
                    agg_hbm.at[cid, pl.ds(row0, ROWS_PER_TILE)])

  return pl.kernel(body, mesh=_sc_mesh(),
                   out_type=jax.ShapeDtypeStruct((NC, NP, D), jnp.float32),
                   scratch_types=scratch)


def _make_sc_counts():
  scratch = [
      pltpu.VMEM((MAXC, CHUNK), jnp.int32),
      pltpu.VMEM((CHUNK, D), jnp.float32),
      pltpu.VMEM((ZROWS, D), jnp.float32),
      pltpu.VMEM_SHARED((NP, D), jnp.float32),
  ]

  def body(e_hbm, zc_hbm, ones_hbm, cnt_hbm, dstv, onesv, zcv, cacc):
    cid = lax.axis_index("c")
    sid = lax.axis_index("s")
    wid = cid * NS + sid
    row0 = sid * ROWS_PER_TILE

    pltpu.sync_copy(zc_hbm, zcv)
    pltpu.sync_copy(ones_hbm, onesv)

    @pl.loop(0, ROWS_PER_TILE // ZROWS)
    def _(j):
      pltpu.sync_copy(zcv, cacc.at[pl.ds(row0 + j * ZROWS, ZROWS)])

    ch0 = wid * MAXC
    pltpu.sync_copy(e_hbm.at[1, pl.ds(ch0, MAXC)], dstv)

    plsc.subcore_barrier()

    @pl.loop(0, MAXC)
    def _(c):
      pltpu.sync_copy(onesv, cacc.at[dstv.at[c]], add=True)

    plsc.subcore_barrier()

    pltpu.sync_copy(cacc.at[pl.ds(row0, ROWS_PER_TILE)],
                    cnt_hbm.at[cid, pl.ds(row0, ROWS_PER_TILE)])

  return pl.kernel(body, mesh=_sc_mesh(),
                   out_type=jax.ShapeDtypeStruct((NC, NP, D), jnp.float32),
                   scratch_types=scratch)


def _sc_agg(h, e3, zrow):
  return _make_sc_agg()(h, e3, zrow)


def _sc_counts(e3, zcnt, ones):
  return _make_sc_counts()(e3, zcnt, ones)


RB = 1000
GRID = N // RB


def _linear_body(agg_ref, cnt_ref, h_ref, wl_ref, bl_ref, wr_ref,
                 y_ref, st_ref, acc_ref):
  i = pl.program_id(0)
  agg = agg_ref[0] + agg_ref[1]
  cnt = cnt_ref[0, :, :1] + cnt_ref[1, :, :1]
  mean = agg / jnp.maximum(cnt, 1.0)
  y = (jnp.dot(mean, wl_ref[...], preferred_element_type=jnp.float32)
       + jnp.dot(h_ref[...], wr_ref[...], preferred_element_type=jnp.float32)
       + bl_ref[...])
  y_ref[...] = y

  @pl.when(i == 0)
  def _():
    acc_ref[...] = jnp.zeros_like(acc_ref)

  acc_ref[0:1, :] += jnp.sum(y, axis=0, keepdims=True)
  acc_ref[1:2, :] += jnp.sum(y * y, axis=0, keepdims=True)
  st_ref[...] = acc_ref[...]


def _linear(aggp, cntp, h, Wl, bl, Wr):
  return pl.pallas_call(
      _linear_body,
      grid=(GRID,),
      in_specs=[
          pl.BlockSpec((NC, RB, D), lambda i: (0, i, 0)),
          pl.BlockSpec((NC, RB, D), lambda i: (0, i, 0)),
          pl.BlockSpec((RB, D), lambda i: (i, 0)),
          pl.BlockSpec((D, H), lambda i: (0, 0)),
          pl.BlockSpec((1, H), lambda i: (0, 0)),
          pl.BlockSpec((D, H), lambda i: (0, 0)),
      ],
      out_specs=[
          pl.BlockSpec((RB, H), lambda i: (i, 0)),
          pl.BlockSpec((8, 128), lambda i: (0, 0)),
      ],
      out_shape=[
          jax.ShapeDtypeStruct((N, H), jnp.float32),
          jax.ShapeDtypeStruct((8, 128), jnp.float32),
      ],
      scratch_shapes=[pltpu.VMEM((8, 128), jnp.float32)],
  )(aggp, cntp, h, Wl, bl, Wr)


def _norm_body(y_ref, st_ref, w_ref, b_ref, h_ref):
  mu = st_ref[0:1, :] * (1.0 / N)
  var = st_ref[1:2, :] * (1.0 / N) - mu * mu
  inv = lax.rsqrt(var + EPS)
  h = (y_ref[...] - mu) * (inv * w_ref[...]) + b_ref[...]
  h_ref[...] = jnp.maximum(h, 0.0)


def _norm_relu(y, st, w, b):
  return pl.pallas_call(
      _norm_body,
      grid=(GRID,),
      in_specs=[
          pl.BlockSpec((RB, H), lambda i: (i, 0)),
          pl.BlockSpec((8, 128), lambda i: (0, 0)),
          pl.BlockSpec((1, H), lambda i: (0, 0)),
          pl.BlockSpec((1, H), lambda i: (0, 0)),
      ],
      out_specs=pl.BlockSpec((RB, H), lambda i: (i, 0)),
      out_shape=jax.ShapeDtypeStruct((N, H), jnp.float32),
  )(y, st, w, b)


def _head_body(y_ref, st_ref, w_ref, b_ref, wc1_ref, bc1_ref, wc2_ref,
               bc2_ref, o_ref):
  mu = st_ref[0:1, :] * (1.0 / N)
  var = st_ref[1:2, :] * (1.0 / N) - mu * mu
  inv = lax.rsqrt(var + EPS)
  h = (y_ref[...] - mu) * (inv * w_ref[...]) + b_ref[...]
  t = jnp.dot(h, wc1_ref[...], preferred_element_type=jnp.float32)
  t = jnp.maximum(t + bc1_ref[...], 0.0)
  o = jnp.sum(t * wc2_ref[...], axis=1, keepdims=True) + bc2_ref[...]
  o_ref[...] = o


def _head(y, st, w, b, Wc1, bc1, wc2r, bc2r):
  return pl.pallas_call(
      _head_body,
      grid=(GRID,),
      in_specs=[
          pl.BlockSpec((RB, H), lambda i: (i, 0)),
          pl.BlockSpec((8, 128), lambda i: (0, 0)),
          pl.BlockSpec((1, H), lambda i: (0, 0)),
          pl.BlockSpec((1, H), lambda i: (0, 0)),
          pl.BlockSpec((H, H // 2), lambda i: (0, 0)),
          pl.BlockSpec((1, H // 2), lambda i: (0, 0)),
          pl.BlockSpec((1, H // 2), lambda i: (0, 0)),
          pl.BlockSpec((1, 1), lambda i: (0, 0)),
      ],
      out_specs=pl.BlockSpec((RB, 1), lambda i: (i, 0)),
      out_shape=jax.ShapeDtypeStruct((N, 1), jnp.float32),
  )(y, st, w, b, Wc1, bc1, wc2r, bc2r)


def kernel(x, edge_index, Wl1, bl1, Wr1, bnw1, bnb1, Wl2, bl2, Wr2, bnw2,
           bnb2, Wl3, bl3, Wr3, bnw3, bnb3, Wc1, bc1, Wc2, bc2):
  pad_src = jnp.zeros((EPAD,), jnp.int32)
  pad_dst = N + (jnp.arange(EPAD, dtype=jnp.int32) % (NP - N))
  e3 = jnp.concatenate(
      [edge_index, jnp.stack([pad_src, pad_dst])], axis=1
  ).reshape(2, NCHUNK, CHUNK)
  zrow = jnp.zeros((ZROWS, D), jnp.float32)
  ones = jnp.ones((CHUNK, D), jnp.float32)

  cntp = _sc_counts(e3, zrow, ones)
  agg1 = _sc_agg(x, e3, zrow)
  y1, st1 = _linear(agg1, cntp, x, Wl1, bl1.reshape(1, H), Wr1)
  h1 = _norm_relu(y1, st1, bnw1.reshape(1, H), bnb1.reshape(1, H))

  agg2 = _sc_agg(h1, e3, zrow)
  y2, st2 = _linear(agg2, cntp, h1, Wl2, bl2.reshape(1, H), Wr2)
  h2 = _norm_relu(y2, st2, bnw2.reshape(1, H), bnb2.reshape(1, H))

  agg3 = _sc_agg(h2, e3, zrow)
  y3, st3 = _linear(agg3, cntp, h2, Wl3, bl3.reshape(1, H), Wr3)
  out = _head(y3, st3, bnw3.reshape(1, H), bnb3.reshape(1, H),
              Wc1, bc1.reshape(1, H // 2), Wc2.reshape(1, H // 2),
              bc2.reshape(1, 1))
  return out[:, 0]
